# baseline (speedup 1.0000x reference)
import functools

import jax
import jax.numpy as jnp
from jax import lax
from jax.experimental import pallas as pl
from jax.experimental.pallas import tpu as pltpu


def _lstm_mlp_kernel(x_ref, wih_ref, whh_ref, bg_ref,
                     w1_ref, b1_ref, w2_ref, b2_ref,
                     out_ref, h_sc, c_sc, xg_sc, hs_sc,
                     *, hp, t_chunk, tb, nc):
    d = x_ref.shape[-1]
    c_id = pl.program_id(1)
    buf = lax.rem(c_id, 2)

    @pl.when(c_id == 0)
    def _():
        h_sc[...] = jnp.zeros_like(h_sc)
        c_sc[...] = jnp.zeros_like(c_sc)

    @pl.when(c_id < nc)
    def _():
        x_flat = x_ref[...].reshape(t_chunk * tb, d)
        xg = jnp.dot(x_flat, wih_ref[...],
                     preferred_element_type=jnp.float32) + bg_ref[...]
        xg_sc[buf] = xg.reshape(t_chunk, tb, 4 * hp).astype(xg_sc.dtype)

    @pl.when(c_id > 0)
    def _():
        whh = whh_ref[...]
        pb = 1 - buf

        def _step(t, carry):
            h_bf, c = carry
            xg_t = xg_sc[pb, t]
            i_g = jax.nn.sigmoid(xg_t[:, 0 * hp:1 * hp] + jnp.dot(
                h_bf, whh[:, 0 * hp:1 * hp],
                preferred_element_type=jnp.float32))
            f_g = jax.nn.sigmoid(xg_t[:, 1 * hp:2 * hp] + jnp.dot(
                h_bf, whh[:, 1 * hp:2 * hp],
                preferred_element_type=jnp.float32))
            g_g = jnp.tanh(xg_t[:, 2 * hp:3 * hp] + jnp.dot(
                h_bf, whh[:, 2 * hp:3 * hp],
                preferred_element_type=jnp.float32))
            o_g = jax.nn.sigmoid(xg_t[:, 3 * hp:4 * hp] + jnp.dot(
                h_bf, whh[:, 3 * hp:4 * hp],
                preferred_element_type=jnp.float32))
            c_new = f_g * c + i_g * g_g
            h_new = (o_g * jnp.tanh(c_new)).astype(jnp.bfloat16)
            hs_sc[t] = h_new
            return h_new, c_new

        h_fin, c_fin = lax.fori_loop(0, t_chunk, _step,
                                     (h_sc[...], c_sc[...]), unroll=8)
        h_sc[...] = h_fin
        c_sc[...] = c_fin

        hsb = hs_sc[...].reshape(t_chunk * tb, hp)
        z = jnp.dot(hsb, w1_ref[...], preferred_element_type=jnp.float32)
        z = jnp.maximum(z + b1_ref[...], 0.0)
        r = jnp.sum(z.reshape(t_chunk, tb, hp) * w2_ref[...], axis=-1)
        out_ref[...] = r + b2_ref[0, 0]


def kernel(x_btd, w_ih, w_hh, b_gates, w1, b1, w2, b2):
    B, T, D = x_btd.shape
    Hp = w_hh.shape[0]

    t_chunk = 64 if (T % 64 == 0) else T
    assert T % t_chunk == 0 and t_chunk % 8 == 0
    tb = B
    nc = T // t_chunk

    body = functools.partial(_lstm_mlp_kernel,
                             hp=Hp, t_chunk=t_chunk, tb=tb, nc=nc)
    rep = lambda shape: pl.BlockSpec(shape, lambda b, c: (0,) * len(shape))

    x_tbd = jnp.transpose(x_btd, (1, 0, 2)).astype(jnp.bfloat16)

    out_tb = pl.pallas_call(
        body,
        out_shape=jax.ShapeDtypeStruct((T, B), jnp.float32),
        grid=(1, nc + 1),
        in_specs=[
            pl.BlockSpec((t_chunk, tb, D),
                         lambda b, c: (jnp.minimum(c, nc - 1), b, 0)),
            rep((D, 4 * Hp)),
            rep((Hp, 4 * Hp)),
            rep((1, 4 * Hp)),
            rep((Hp, Hp)),
            rep((1, Hp)),
            rep((1, Hp)),
            rep((1, 1)),
        ],
        out_specs=pl.BlockSpec((t_chunk, tb),
                               lambda b, c: (jnp.maximum(c - 1, 0), b)),
        scratch_shapes=[
            pltpu.VMEM((tb, Hp), jnp.bfloat16),
            pltpu.VMEM((tb, Hp), jnp.float32),
            pltpu.VMEM((2, t_chunk, tb, 4 * Hp), jnp.bfloat16),
            pltpu.VMEM((t_chunk, tb, Hp), jnp.bfloat16),
        ],
        compiler_params=pltpu.CompilerParams(
            dimension_semantics=("parallel", "arbitrary"),
            vmem_limit_bytes=63 * 1024 * 1024,
        ),
    )(x_tbd, w_ih, w_hh, b_gates, w1, b1, w2, b2)

    return jnp.transpose(out_tb)[..., None]
